# speedup vs baseline: 13.9142x; 13.9142x over previous
import jax
import jax.numpy as jnp
from jax import lax
from jax.experimental import pallas as pl
from jax.experimental.pallas import tpu as pltpu

_N = 500000
_C = 14
_D = 64
_PAD = 15
_B = 2000


def _body(s_ref, rp_ref, a_ref, ap_ref, res_ref, atom_ref, apos_ref,
          sinc_ref, out_ref):
    f32 = jnp.float32
    s = s_ref[...]
    rp = rp_ref[...].astype(f32)
    a = a_ref[...]
    ap = ap_ref[...]
    B = s.shape[0]

    oh_s = (s == lax.broadcasted_iota(jnp.int32, (1, 32), 1)).astype(f32)
    left = jnp.dot(oh_s, res_ref[...], preferred_element_type=f32)
    freq = sinc_ref[0:1, :]
    phase = sinc_ref[1:2, :]
    left = left + jnp.sin(rp * freq + phase)

    acc_a = jnp.zeros((B, 40), f32)
    acc_p = jnp.zeros((B, 16), f32)
    iota40 = lax.broadcasted_iota(jnp.int32, (1, 40), 1)
    iota16 = lax.broadcasted_iota(jnp.int32, (1, 16), 1)
    for c in range(_C):
        ac = a[:, c:c + 1]
        apc = ap[:, c:c + 1]
        m = apc != _PAD
        acc_a = acc_a + ((ac == iota40) & m).astype(f32)
        acc_p = acc_p + (apc == iota16).astype(f32)
    cnt = jnp.float32(_C) - acc_p[:, _PAD:_PAD + 1]
    right = (jnp.dot(acc_a, atom_ref[...], preferred_element_type=f32)
             + jnp.dot(acc_p, apos_ref[...], preferred_element_type=f32))
    right = right / (cnt + 1e-10)

    out_ref[:, 0:_D] = left
    out_ref[:, _D:2 * _D] = right


def kernel(S, RP, A, AP, res_table, atom_table, atom_pos_table):
    f32 = jnp.float32
    n = S.shape[0]
    assert n % _B == 0
    grid = n // _B

    res_pad = jnp.zeros((32, _D), f32).at[:25].set(res_table)
    atom_pad = jnp.zeros((40, _D), f32).at[:38].set(atom_table)
    apos_z = atom_pos_table.at[_PAD].set(0.0)
    k = jnp.arange(_D)
    f2 = jnp.power(10000.0, -2.0 * (k // 2).astype(f32) / _D)
    ph = (k % 2).astype(f32) * jnp.float32(jnp.pi / 2)
    sinc = jnp.zeros((8, _D), f32).at[0].set(f2).at[1].set(ph)

    out = pl.pallas_call(
        _body,
        grid=(grid,),
        in_specs=[
            pl.BlockSpec((_B, 1), lambda i: (i, 0)),
            pl.BlockSpec((_B, 1), lambda i: (i, 0)),
            pl.BlockSpec((_B, _C), lambda i: (i, 0)),
            pl.BlockSpec((_B, _C), lambda i: (i, 0)),
            pl.BlockSpec((32, _D), lambda i: (0, 0)),
            pl.BlockSpec((40, _D), lambda i: (0, 0)),
            pl.BlockSpec((16, _D), lambda i: (0, 0)),
            pl.BlockSpec((8, _D), lambda i: (0, 0)),
        ],
        out_specs=pl.BlockSpec((_B, 2 * _D), lambda i: (i, 0)),
        out_shape=jax.ShapeDtypeStruct((n, 2 * _D), f32),
    )(S[:, None], RP[:, None], A, AP, res_pad, atom_pad, apos_z, sinc)
    return out


# speedup vs baseline: 44.8614x; 3.2241x over previous
import functools

import jax
import jax.numpy as jnp
from jax import lax
from jax.experimental import pallas as pl
from jax.experimental.pallas import tpu as pltpu
from jax.experimental.pallas import tpu_sc as plsc

_C = 14
_D = 64
_PAD = 15
_F = 96
_R = 800
_NW = 32
_B = 2000


def _sc_hist(n, s_hbm, a_hbm, ap_hbm, feat_hbm, a_v, ap_v, s_v, feat_v):
    f32 = jnp.float32
    i32 = jnp.int32
    nchunks = n // _R
    kmax = (nchunks + _NW - 1) // _NW
    wid = lax.axis_index("s") * 2 + lax.axis_index("c")
    iota = lax.iota(i32, 16)
    idx14 = iota * _C
    idxf = iota * _F
    ones = jnp.ones((16,), f32)
    zeros = jnp.zeros((16,), f32)

    def chunk_body(k, carry):
        cid = wid + k * _NW

        @pl.when(cid < nchunks)
        def _():
            base = cid * _R
            pltpu.sync_copy(a_hbm.at[pl.ds(base * _C, _R * _C)], a_v)
            pltpu.sync_copy(ap_hbm.at[pl.ds(base * _C, _R * _C)], ap_v)
            pltpu.sync_copy(s_hbm.at[pl.ds(base, _R)], s_v)

            def zbody(i, c2):
                for u in range(16):
                    feat_v[pl.ds(i * 256 + u * 16, 16)] = zeros
                return c2

            lax.fori_loop(0, _R * _F // 256, zbody, 0)

            def gbody(g, c2):
                apb = g * (16 * _C)
                aps = []
                cnt = zeros
                for c in range(_C):
                    apc = plsc.load_gather(ap_v, [idx14 + (apb + c)])
                    aps.append(apc)
                    cnt = cnt + jnp.where(apc != _PAD, 1.0, 0.0)
                recip = 1.0 / (cnt + 1e-10)
                fb = g * (16 * _F) + idxf
                for c in range(_C):
                    ac = plsc.load_gather(a_v, [idx14 + (apb + c)])
                    plsc.addupdate_scatter(feat_v, [fb + ac], recip,
                                           mask=aps[c] != _PAD)
                    plsc.addupdate_scatter(feat_v, [fb + (aps[c] + 40)],
                                           recip)
                sv = plsc.load_gather(s_v, [iota + g * 16])
                plsc.addupdate_scatter(feat_v, [fb + (sv + 56)], ones)
                return c2

            lax.fori_loop(0, _R // 16, gbody, 0)
            pltpu.sync_copy(feat_v, feat_hbm.at[pl.ds(base * _F, _R * _F)])

        return carry

    lax.fori_loop(0, kmax, chunk_body, 0)


def _tc_body(feat_ref, rp_ref, t_ref, st_ref, out_ref):
    f32 = jnp.float32
    full = jnp.dot(feat_ref[...], t_ref[...], preferred_element_type=f32)
    rp = rp_ref[...].astype(f32)
    rpb = jnp.dot(rp, jnp.ones((1, 256), f32), preferred_element_type=f32)
    oh = (rpb == lax.broadcasted_iota(jnp.int32, (1, 256), 1).astype(f32))
    sinrows = jnp.dot(oh.astype(f32), st_ref[...], preferred_element_type=f32)
    out_ref[:, 0:_D] = full[:, 0:_D] + sinrows
    out_ref[:, _D:2 * _D] = full[:, _D:2 * _D]


def kernel(S, RP, A, AP, res_table, atom_table, atom_pos_table):
    f32 = jnp.float32
    n = S.shape[0]
    assert n % _R == 0 and n % _B == 0 and _R % 8 == 0

    mesh = plsc.VectorSubcoreMesh(core_axis_name="c", subcore_axis_name="s")
    sc = pl.kernel(
        functools.partial(_sc_hist, n),
        out_type=jax.ShapeDtypeStruct((n * _F,), f32),
        mesh=mesh,
        compiler_params=pltpu.CompilerParams(needs_layout_passes=False),
        scratch_types=[
            pltpu.VMEM((_R * _C,), jnp.int32),
            pltpu.VMEM((_R * _C,), jnp.int32),
            pltpu.VMEM((_R,), jnp.int32),
            pltpu.VMEM((_R * _F,), f32),
        ],
    )
    feat = sc(S, A.reshape(-1), AP.reshape(-1)).reshape(n, _F)

    T = jnp.zeros((_F, 2 * _D), f32)
    T = T.at[0:38, _D:2 * _D].set(atom_table)
    T = T.at[40:56, _D:2 * _D].set(atom_pos_table)
    T = T.at[40 + _PAD, :].set(0.0)
    T = T.at[56:81, 0:_D].set(res_table)
    pos = jnp.arange(256, dtype=f32)[:, None]
    idx = jnp.power(10000.0,
                    -2.0 * jnp.arange(_D // 2, dtype=f32) / _D)[None, :]
    emb = pos * idx
    st = jnp.stack([jnp.sin(emb), jnp.cos(emb)], axis=-1).reshape(256, _D)

    out = pl.pallas_call(
        _tc_body,
        grid=(n // _B,),
        in_specs=[
            pl.BlockSpec((_B, _F), lambda i: (i, 0)),
            pl.BlockSpec((_B, 1), lambda i: (i, 0)),
            pl.BlockSpec((_F, 2 * _D), lambda i: (0, 0)),
            pl.BlockSpec((256, _D), lambda i: (0, 0)),
        ],
        out_specs=pl.BlockSpec((_B, 2 * _D), lambda i: (i, 0)),
        out_shape=jax.ShapeDtypeStruct((n, 2 * _D), f32),
    )(feat, RP[:, None], T, st)
    return out
